# baseline (speedup 1.0000x reference)
import functools

import jax
import jax.numpy as jnp
from jax import lax
from jax.experimental import pallas as pl
from jax.experimental.pallas import tpu as pltpu
from jax.experimental.pallas import tpu_sc as plsc

LANES = 16
CHUNK = 64


def _vec_sqrt(x):
    xg = jnp.maximum(x, jnp.float32(1e-35))
    i = lax.bitcast_convert_type(xg, jnp.int32)
    i = jnp.int32(0x5F3759DF) - lax.shift_right_logical(i, jnp.int32(1))
    y = lax.bitcast_convert_type(i, jnp.float32)
    half = jnp.float32(0.5) * xg
    for _ in range(3):
        y = y * (jnp.float32(1.5) - half * y * y)
    return x * y


def _make_transe(B, D):
    info = plsc.get_sparse_core_info()
    NC, NS = info.num_cores, info.num_subcores
    NW = NC * NS
    per_w = B // NW
    n_chunks = per_w // CHUNK
    D2 = 2 * D
    assert per_w % CHUNK == 0 and D % LANES == 0

    mesh = plsc.VectorSubcoreMesh(core_axis_name="c", subcore_axis_name="s")

    row_buf = pltpu.VMEM((CHUNK, D2), jnp.float32)
    idx_buf = pltpu.VMEM((per_w,), jnp.int32)

    @functools.partial(
        pl.kernel,
        mesh=mesh,
        compiler_params=pltpu.CompilerParams(needs_layout_passes=False),
        out_type=(
            jax.ShapeDtypeStruct((B,), jnp.float32),
            jax.ShapeDtypeStruct((B,), jnp.float32),
        ),
        scratch_types=[
            idx_buf, idx_buf, idx_buf, idx_buf, idx_buf,
            idx_buf, idx_buf, idx_buf, idx_buf, idx_buf,
            row_buf, row_buf, row_buf, row_buf, row_buf,
            row_buf, row_buf, row_buf, row_buf, row_buf,
            pltpu.VMEM((per_w,), jnp.float32),
            pltpu.VMEM((per_w,), jnp.float32),
            pltpu.SemaphoreType.DMA,
            pltpu.SemaphoreType.DMA,
            pltpu.SemaphoreType.DMA,
        ],
    )
    def transe(ph_idx, pt_idx, nh_idx, nt_idx, r_idx, ent2, rel2,
               pos_out, neg_out,
               rph, rpt, rnh, rnt, rrl,
               tph, tpt, tnh, tnt, trl,
               ph0, pt0, nh0, nt0, rr0,
               ph1, pt1, nh1, nt1, rr1,
               po, no, sem_i, sem0, sem1):
        wid = lax.axis_index("s") * NC + lax.axis_index("c")
        base_w = wid * per_w
        raws = (rph, rpt, rnh, rnt, rrl)
        tids = (tph, tpt, tnh, tnt, trl)
        bufs = ((ph0, pt0, nh0, nt0, rr0), (ph1, pt1, nh1, nt1, rr1))
        sems = (sem0, sem1)

        idx_cps = [
            pltpu.async_copy(src.at[pl.ds(base_w, per_w)], dst, sem_i)
            for src, dst in zip((ph_idx, pt_idx, nh_idx, nt_idx, r_idx), raws)
        ]
        for cp in idx_cps:
            cp.wait()

        def shift_body(i, _):
            sl = pl.ds(i * LANES, LANES)
            for raw, tid in zip(raws, tids):
                tid[sl] = lax.shift_right_logical(raw[sl], jnp.int32(1))
            return 0

        lax.fori_loop(0, per_w // LANES, shift_body, 0)

        def fire(c, par):
            sl = pl.ds(c * CHUNK, CHUNK)
            sem = sems[par]
            cps = []
            for tid, dst in zip(tids[:4], bufs[par][:4]):
                cps.append(pltpu.async_copy(ent2.at[tid.at[sl]], dst, sem))
            cps.append(pltpu.async_copy(rel2.at[trl.at[sl]], bufs[par][4], sem))
            return cps

        lane_ids = lax.iota(jnp.int32, LANES)
        one = jnp.int32(1)
        in_flight = fire(0, 0)
        for c in range(n_chunks):
            par = c & 1
            for cp in in_flight:
                cp.wait()
            if c + 1 < n_chunks:
                in_flight = fire(c + 1, 1 - par)
            bset = bufs[par]
            out0 = c * CHUNK

            def group_body(g, _):
                row0 = g * LANES
                pvs = [(raw[pl.ds(out0 + row0, LANES)] & one) * jnp.int32(D)
                       for raw in raws]
                pvec = jnp.zeros((LANES,), jnp.float32)
                nvec = jnp.zeros((LANES,), jnp.float32)
                for j in range(LANES):
                    r = row0 + j
                    offs = [pv[j] for pv in pvs]
                    pacc = jnp.zeros((LANES,), jnp.float32)
                    nacc = jnp.zeros((LANES,), jnp.float32)
                    for d in range(D // LANES):
                        hv = bset[0][r, pl.ds(offs[0] + d * LANES, LANES)]
                        tv = bset[1][r, pl.ds(offs[1] + d * LANES, LANES)]
                        nhv = bset[2][r, pl.ds(offs[2] + d * LANES, LANES)]
                        ntv = bset[3][r, pl.ds(offs[3] + d * LANES, LANES)]
                        rv = bset[4][r, pl.ds(offs[4] + d * LANES, LANES)]
                        pd = hv + rv - tv
                        nd = nhv + rv - ntv
                        pacc = pacc + pd * pd
                        nacc = nacc + nd * nd
                    jmask = lane_ids == j
                    pvec = jnp.where(jmask, jnp.sum(pacc), pvec)
                    nvec = jnp.where(jmask, jnp.sum(nacc), nvec)
                po[pl.ds(out0 + row0, LANES)] = _vec_sqrt(pvec)
                no[pl.ds(out0 + row0, LANES)] = _vec_sqrt(nvec)
                return 0

            lax.fori_loop(0, CHUNK // LANES, group_body, 0)

        pltpu.sync_copy(po, pos_out.at[pl.ds(base_w, per_w)])
        pltpu.sync_copy(no, neg_out.at[pl.ds(base_w, per_w)])

    return transe


def kernel(pos_edge_index, edge_type, neg_edge_index, entity_embeddings,
           relation_embeddings):
    B = pos_edge_index.shape[1]
    E, D = entity_embeddings.shape
    R = relation_embeddings.shape[0]
    ent2 = entity_embeddings.reshape(E // 2, 2 * D)
    rel2 = relation_embeddings.reshape(R // 2, 2 * D)
    fn = _make_transe(B, D)
    return fn(pos_edge_index[0], pos_edge_index[1],
              neg_edge_index[0], neg_edge_index[1], edge_type, ent2, rel2)


# speedup vs baseline: 1.6037x; 1.6037x over previous
import functools

import jax
import jax.numpy as jnp
from jax import lax
from jax.experimental import pallas as pl
from jax.experimental.pallas import tpu as pltpu
from jax.experimental.pallas import tpu_sc as plsc

LANES = 16
CHUNK = 64


def _vec_sqrt(x):
    xg = jnp.maximum(x, jnp.float32(1e-35))
    i = lax.bitcast_convert_type(xg, jnp.int32)
    i = jnp.int32(0x5F3759DF) - lax.shift_right_logical(i, jnp.int32(1))
    y = lax.bitcast_convert_type(i, jnp.float32)
    half = jnp.float32(0.5) * xg
    for _ in range(3):
        y = y * (jnp.float32(1.5) - half * y * y)
    return x * y


def _make_transe(B, D):
    info = plsc.get_sparse_core_info()
    NC, NS = info.num_cores, info.num_subcores
    NW = NC * NS
    per_w = B // NW
    n_chunks = per_w // CHUNK
    assert per_w % CHUNK == 0 and D % LANES == 0

    mesh = plsc.VectorSubcoreMesh(core_axis_name="c", subcore_axis_name="s")

    row_buf = pltpu.VMEM((CHUNK, D), jnp.float32)
    idx_buf = pltpu.VMEM((per_w,), jnp.int32)

    @functools.partial(
        pl.kernel,
        mesh=mesh,
        compiler_params=pltpu.CompilerParams(needs_layout_passes=False),
        out_type=(
            jax.ShapeDtypeStruct((B,), jnp.float32),
            jax.ShapeDtypeStruct((B,), jnp.float32),
        ),
        scratch_types=[
            idx_buf, idx_buf, idx_buf, idx_buf, idx_buf,
            row_buf, row_buf, row_buf, row_buf, row_buf,
            row_buf, row_buf, row_buf, row_buf, row_buf,
            pltpu.VMEM((per_w,), jnp.float32),
            pltpu.VMEM((per_w,), jnp.float32),
            pltpu.SemaphoreType.DMA,
            pltpu.SemaphoreType.DMA,
            pltpu.SemaphoreType.DMA,
        ],
    )
    def transe(ph_idx, pt_idx, nh_idx, nt_idx, r_idx, ent, rel,
               pos_out, neg_out,
               rph, rpt, rnh, rnt, rrl,
               ph0, pt0, nh0, nt0, rr0,
               ph1, pt1, nh1, nt1, rr1,
               po, no, sem_i, sem0, sem1):
        wid = lax.axis_index("s") * NC + lax.axis_index("c")
        base_w = wid * per_w
        raws = (rph, rpt, rnh, rnt, rrl)
        bufs = ((ph0, pt0, nh0, nt0, rr0), (ph1, pt1, nh1, nt1, rr1))
        sems = (sem0, sem1)

        idx_cps = [
            pltpu.async_copy(src.at[pl.ds(base_w, per_w)], dst, sem_i)
            for src, dst in zip((ph_idx, pt_idx, nh_idx, nt_idx, r_idx), raws)
        ]
        for cp in idx_cps:
            cp.wait()

        def fire(c, par):
            sem = sems[par]
            bset = bufs[par]

            def fire_group(g, _):
                r0 = c * CHUNK + g * LANES
                for raw, dst, tab in zip(
                        raws, bset, (ent, ent, ent, ent, rel)):
                    iv = raw[pl.ds(r0, LANES)]
                    for j in range(LANES):
                        row = iv[j]
                        dr = g * LANES + j
                        pltpu.async_copy(tab.at[row], dst.at[dr], sem)
                return 0

            lax.fori_loop(0, CHUNK // LANES, fire_group, 0)

        def drain(par):
            def drain_one(i, _):
                pltpu.make_async_copy(
                    ent.at[0], bufs[par][0].at[0], sems[par]).wait()
                return 0

            lax.fori_loop(0, 5 * CHUNK, drain_one, 0)

        lane_ids = lax.iota(jnp.int32, LANES)
        fire(0, 0)
        for c in range(n_chunks):
            par = c & 1
            drain(par)
            if c + 1 < n_chunks:
                fire(c + 1, 1 - par)
            bset = bufs[par]
            out0 = c * CHUNK

            def group_body(g, _):
                row0 = g * LANES
                pvec = jnp.zeros((LANES,), jnp.float32)
                nvec = jnp.zeros((LANES,), jnp.float32)
                for j in range(LANES):
                    r = row0 + j
                    pacc = jnp.zeros((LANES,), jnp.float32)
                    nacc = jnp.zeros((LANES,), jnp.float32)
                    for d in range(D // LANES):
                        sl = pl.ds(d * LANES, LANES)
                        rv = bset[4][r, sl]
                        pd = bset[0][r, sl] + rv - bset[1][r, sl]
                        nd = bset[2][r, sl] + rv - bset[3][r, sl]
                        pacc = pacc + pd * pd
                        nacc = nacc + nd * nd
                    jmask = lane_ids == j
                    pvec = jnp.where(jmask, jnp.sum(pacc), pvec)
                    nvec = jnp.where(jmask, jnp.sum(nacc), nvec)
                po[pl.ds(out0 + row0, LANES)] = _vec_sqrt(pvec)
                no[pl.ds(out0 + row0, LANES)] = _vec_sqrt(nvec)
                return 0

            lax.fori_loop(0, CHUNK // LANES, group_body, 0)

        pltpu.sync_copy(po, pos_out.at[pl.ds(base_w, per_w)])
        pltpu.sync_copy(no, neg_out.at[pl.ds(base_w, per_w)])

    return transe


def kernel(pos_edge_index, edge_type, neg_edge_index, entity_embeddings,
           relation_embeddings):
    B = pos_edge_index.shape[1]
    D = entity_embeddings.shape[1]
    fn = _make_transe(B, D)
    return fn(pos_edge_index[0], pos_edge_index[1],
              neg_edge_index[0], neg_edge_index[1], edge_type,
              entity_embeddings, relation_embeddings)
